# baseline (speedup 1.0000x reference)
import functools

import jax
import jax.numpy as jnp
from jax import lax
from jax.experimental import pallas as pl
from jax.experimental.pallas import tpu as pltpu
from jax.experimental.pallas import tpu_sc as plsc

D = 64
B = 4096
MARGIN = 1.0

_NC = 2
_NS = 16
_NW = _NC * _NS
_JOBS = 2 * B
_JPW = _JOBS // _NW
_CH = 16
_NCHUNK = _JPW // _CH


def _allsum16(v):
    iota = lax.iota(jnp.int32, 16)
    for sh in (1, 2, 4, 8):
        v = v + v.at[iota ^ sh].get(mode="promise_in_bounds")
    return v


def _rsqrt16(x):
    i = lax.bitcast_convert_type(x, jnp.int32)
    y = lax.bitcast_convert_type(jnp.int32(0x5F3759DF) - (i >> 1), jnp.float32)
    for _ in range(3):
        y = y * (1.5 - 0.5 * x * y * y)
    return y


def _sc_body(hid_hbm, tid_hbm, rid_hbm, ent2_hbm, rel_hbm, tr_hbm, out_hbm,
             hid_v, tid_v, rid_v, m_v, h_v, t_v, r_v, acc_v, sem):
    wid = lax.axis_index("s") * _NC + lax.axis_index("c")
    base = wid * _JPW
    pltpu.sync_copy(hid_hbm.at[pl.ds(base, _JPW)], hid_v)
    pltpu.sync_copy(tid_hbm.at[pl.ds(base, _JPW)], tid_v)
    pltpu.sync_copy(rid_hbm.at[pl.ds(base, _JPW)], rid_v)

    splats = [jnp.full((16,), i, jnp.int32) for i in range(16)]

    def job_norms(acc):
        ss = acc[0] * acc[0] + acc[1] * acc[1] + acc[2] * acc[2] + acc[3] * acc[3]
        return _rsqrt16(jnp.maximum(_allsum16(ss), 1e-12))

    def job_scores(j, ah, at):
        inv_h = job_norms(ah)
        inv_t = job_norms(at)
        s_abs = jnp.zeros((16,), jnp.float32)
        s_sq = jnp.zeros((16,), jnp.float32)
        for k in range(4):
            rk = r_v[j, pl.ds(16 * k, 16)]
            dk = inv_h * ah[k] + rk - inv_t * at[k]
            s_abs = s_abs + jnp.abs(dk)
            s_sq = s_sq + dk * dk
        return _allsum16(s_abs), _allsum16(s_sq)

    def chunk_body(c, carry):
        acc1_o, acc2_o = carry
        cb = c * _CH
        hidx = hid_v[pl.ds(cb, _CH)]
        tidx = tid_v[pl.ds(cb, _CH)]
        ridx = rid_v[pl.ds(cb, _CH)]
        cps = [pltpu.async_copy(tr_hbm.at[ridx], m_v, sem)]
        for jj in range(_CH):
            cps.append(pltpu.async_copy(
                ent2_hbm.at[hidx[jj] >> 1], h_v.at[jj], sem))
            cps.append(pltpu.async_copy(
                ent2_hbm.at[tidx[jj] >> 1], t_v.at[jj], sem))
            cps.append(pltpu.async_copy(rel_hbm.at[ridx[jj]], r_v.at[jj], sem))
        for cp in cps:
            cp.wait()

        def pair_body(p, pc):
            acc1, acc2 = pc
            j0 = 2 * p
            j1 = j0 + 1
            s0 = jnp.full((16,), j0, jnp.int32)
            s1 = jnp.full((16,), j1, jnp.int32)
            mh0 = (hidx.at[s0].get(mode="promise_in_bounds") & 1).astype(jnp.float32)
            mh1 = (hidx.at[s1].get(mode="promise_in_bounds") & 1).astype(jnp.float32)
            mt0 = (tidx.at[s0].get(mode="promise_in_bounds") & 1).astype(jnp.float32)
            mt1 = (tidx.at[s1].get(mode="promise_in_bounds") & 1).astype(jnp.float32)
            aph = [jnp.zeros((16,), jnp.float32) for _ in range(4)]
            apt = [jnp.zeros((16,), jnp.float32) for _ in range(4)]
            anh = [jnp.zeros((16,), jnp.float32) for _ in range(4)]
            ant = [jnp.zeros((16,), jnp.float32) for _ in range(4)]
            for d16 in range(4):
                lo = d16 * 16
                hi = D + d16 * 16
                hpv0 = h_v[j0, pl.ds(lo, 16)]
                hpv = hpv0 + mh0 * (h_v[j0, pl.ds(hi, 16)] - hpv0)
                hnv0 = h_v[j1, pl.ds(lo, 16)]
                hnv = hnv0 + mh1 * (h_v[j1, pl.ds(hi, 16)] - hnv0)
                tpv0 = t_v[j0, pl.ds(lo, 16)]
                tpv = tpv0 + mt0 * (t_v[j0, pl.ds(hi, 16)] - tpv0)
                tnv0 = t_v[j1, pl.ds(lo, 16)]
                tnv = tnv0 + mt1 * (t_v[j1, pl.ds(hi, 16)] - tnv0)
                for dd in range(16):
                    hp = hpv.at[splats[dd]].get(mode="promise_in_bounds")
                    hn = hnv.at[splats[dd]].get(mode="promise_in_bounds")
                    tp = tpv.at[splats[dd]].get(mode="promise_in_bounds")
                    tn = tnv.at[splats[dd]].get(mode="promise_in_bounds")
                    off = (d16 * 16 + dd) * D
                    for k in range(4):
                        mp = m_v[j0, pl.ds(off + 16 * k, 16)]
                        mn = m_v[j1, pl.ds(off + 16 * k, 16)]
                        aph[k] = aph[k] + hp * mp
                        apt[k] = apt[k] + tp * mp
                        anh[k] = anh[k] + hn * mn
                        ant[k] = ant[k] + tn * mn
            p1, p2 = job_scores(j0, aph, apt)
            n1, n2 = job_scores(j1, anh, ant)
            acc1 = acc1 + jnp.maximum(p1 - n1 + MARGIN, 0.0)
            acc2 = acc2 + jnp.maximum(p2 - n2 + MARGIN, 0.0)
            return acc1, acc2

        return lax.fori_loop(0, _CH // 2, pair_body, (acc1_o, acc2_o))

    zero16 = jnp.zeros((16,), jnp.float32)
    acc1, acc2 = lax.fori_loop(0, _NCHUNK, chunk_body, (zero16, zero16))

    lane = lax.iota(jnp.int32, 16)
    res = jnp.where(lane == 0, acc1, jnp.where(lane == 1, acc2, 0.0))
    acc_v[...] = res
    pltpu.sync_copy(acc_v, out_hbm.at[wid])


def _run_sc(x, ent_emb, rel_emb, transfer):
    pos_h, pos_t, pos_r = x[:, 0], x[:, 1], x[:, 2]
    neg_h, neg_t, neg_r = x[:, 3], x[:, 4], x[:, 5]
    h_ids = jnp.stack([pos_h, neg_h], axis=1).reshape(-1)
    t_ids = jnp.stack([pos_t, neg_t], axis=1).reshape(-1)
    r_ids = jnp.stack([pos_r, neg_r], axis=1).reshape(-1)

    ent2 = ent_emb.reshape(-1, 2 * D)

    mesh = plsc.VectorSubcoreMesh(core_axis_name="c", subcore_axis_name="s")
    run = functools.partial(
        pl.kernel,
        out_type=jax.ShapeDtypeStruct((_NW, 16), jnp.float32),
        mesh=mesh,
        scratch_types=[
            pltpu.VMEM((_JPW,), jnp.int32),
            pltpu.VMEM((_JPW,), jnp.int32),
            pltpu.VMEM((_JPW,), jnp.int32),
            pltpu.VMEM((_CH, D * D), jnp.float32),
            pltpu.VMEM((_CH, 2 * D), jnp.float32),
            pltpu.VMEM((_CH, 2 * D), jnp.float32),
            pltpu.VMEM((_CH, D), jnp.float32),
            pltpu.VMEM((16,), jnp.float32),
            pltpu.SemaphoreType.DMA,
        ],
    )(_sc_body)
    return run(h_ids, t_ids, r_ids, ent2, rel_emb, transfer)


def kernel(x, ent_emb, rel_emb, transfer, l1_flag):
    part = _run_sc(x, ent_emb, rel_emb, transfer)
    loss1 = jnp.sum(part[:, 0])
    loss2 = jnp.sum(part[:, 1])
    return jnp.where(l1_flag, loss1, loss2)


# speedup vs baseline: 1.4566x; 1.4566x over previous
import functools

import jax
import jax.numpy as jnp
from jax import lax
from jax.experimental import pallas as pl
from jax.experimental.pallas import tpu as pltpu
from jax.experimental.pallas import tpu_sc as plsc

D = 64
B = 4096
MARGIN = 1.0

_NC = 2
_NS = 16
_NW = _NC * _NS
_JOBS = 2 * B
_JPW = _JOBS // _NW
_CH = 16
_NCHUNK = _JPW // _CH


def _allsum16(v):
    iota = lax.iota(jnp.int32, 16)
    for sh in (1, 2, 4, 8):
        v = v + v.at[iota ^ sh].get(mode="promise_in_bounds")
    return v


def _rsqrt16(x):
    i = lax.bitcast_convert_type(x, jnp.int32)
    y = lax.bitcast_convert_type(jnp.int32(0x5F3759DF) - (i >> 1), jnp.float32)
    for _ in range(3):
        y = y * (1.5 - 0.5 * x * y * y)
    return y


def _sc_body(hid_hbm, tid_hbm, rid_hbm, ent_hbm, rel_hbm, tr_hbm, out_hbm,
             hid_v, tid_v, rid_v, m_v, h_v, t_v, r_v, acc_v,
             sem_a, sem_b, sem_s):
    wid = lax.axis_index("s") * _NC + lax.axis_index("c")
    base = wid * _JPW
    pltpu.sync_copy(hid_hbm.at[pl.ds(base, _JPW)], hid_v)
    pltpu.sync_copy(tid_hbm.at[pl.ds(base, _JPW)], tid_v)
    pltpu.sync_copy(rid_hbm.at[pl.ds(base, _JPW)], rid_v)

    splats = [jnp.full((16,), i, jnp.int32) for i in range(16)]

    def job_norms(acc):
        ss = acc[0] * acc[0] + acc[1] * acc[1] + acc[2] * acc[2] + acc[3] * acc[3]
        return _rsqrt16(jnp.maximum(_allsum16(ss), 1e-12))

    def job_scores(j, ah, at):
        inv_h = job_norms(ah)
        inv_t = job_norms(at)
        s_abs = jnp.zeros((16,), jnp.float32)
        s_sq = jnp.zeros((16,), jnp.float32)
        for k in range(4):
            rk = r_v[j, pl.ds(16 * k, 16)]
            dk = inv_h * ah[k] + rk - inv_t * at[k]
            s_abs = s_abs + jnp.abs(dk)
            s_sq = s_sq + dk * dk
        return _allsum16(s_abs), _allsum16(s_sq)

    def pair_body(p, pc):
        acc1, acc2 = pc
        j0 = 2 * p
        j1 = j0 + 1
        aph = [jnp.zeros((16,), jnp.float32) for _ in range(4)]
        apt = [jnp.zeros((16,), jnp.float32) for _ in range(4)]
        anh = [jnp.zeros((16,), jnp.float32) for _ in range(4)]
        ant = [jnp.zeros((16,), jnp.float32) for _ in range(4)]
        for d16 in range(4):
            hpv = h_v[j0, pl.ds(d16 * 16, 16)]
            hnv = h_v[j1, pl.ds(d16 * 16, 16)]
            tpv = t_v[j0, pl.ds(d16 * 16, 16)]
            tnv = t_v[j1, pl.ds(d16 * 16, 16)]
            for dd in range(16):
                hp = hpv.at[splats[dd]].get(mode="promise_in_bounds")
                hn = hnv.at[splats[dd]].get(mode="promise_in_bounds")
                tp = tpv.at[splats[dd]].get(mode="promise_in_bounds")
                tn = tnv.at[splats[dd]].get(mode="promise_in_bounds")
                off = (d16 * 16 + dd) * D
                for k in range(4):
                    mp = m_v[j0, pl.ds(off + 16 * k, 16)]
                    mn = m_v[j1, pl.ds(off + 16 * k, 16)]
                    aph[k] = aph[k] + hp * mp
                    apt[k] = apt[k] + tp * mp
                    anh[k] = anh[k] + hn * mn
                    ant[k] = ant[k] + tn * mn
        p1, p2 = job_scores(j0, aph, apt)
        n1, n2 = job_scores(j1, anh, ant)
        acc1 = acc1 + jnp.maximum(p1 - n1 + MARGIN, 0.0)
        acc2 = acc2 + jnp.maximum(p2 - n2 + MARGIN, 0.0)
        return acc1, acc2

    def chunk_body(c, carry):
        cb = c * _CH
        hidx = hid_v[pl.ds(cb, _CH)]
        tidx = tid_v[pl.ds(cb, _CH)]
        ridx = rid_v[pl.ds(cb, _CH)]
        cp_a = pltpu.async_copy(
            tr_hbm.at[rid_v.at[pl.ds(cb, _CH // 2)]], m_v.at[pl.ds(0, _CH // 2)],
            sem_a)
        cp_b = pltpu.async_copy(
            tr_hbm.at[rid_v.at[pl.ds(cb + _CH // 2, _CH // 2)]],
            m_v.at[pl.ds(_CH // 2, _CH // 2)], sem_b)
        cps = []
        for jj in range(_CH):
            cps.append(pltpu.async_copy(ent_hbm.at[hidx[jj]], h_v.at[jj], sem_s))
            cps.append(pltpu.async_copy(ent_hbm.at[tidx[jj]], t_v.at[jj], sem_s))
            cps.append(pltpu.async_copy(rel_hbm.at[ridx[jj]], r_v.at[jj], sem_s))
        for cp in cps:
            cp.wait()
        cp_a.wait()
        half1 = lax.fori_loop(0, _CH // 4, pair_body, carry)
        cp_b.wait()
        return lax.fori_loop(_CH // 4, _CH // 2, pair_body, half1)

    zero16 = jnp.zeros((16,), jnp.float32)
    acc1, acc2 = lax.fori_loop(0, _NCHUNK, chunk_body, (zero16, zero16))

    lane = lax.iota(jnp.int32, 16)
    res = jnp.where(lane == 0, acc1, jnp.where(lane == 1, acc2, 0.0))
    acc_v[...] = res
    pltpu.sync_copy(acc_v, out_hbm.at[wid])


def _run_sc(x, ent_emb, rel_emb, transfer):
    pos_h, pos_t, pos_r = x[:, 0], x[:, 1], x[:, 2]
    neg_h, neg_t, neg_r = x[:, 3], x[:, 4], x[:, 5]
    h_ids = jnp.stack([pos_h, neg_h], axis=1).reshape(-1)
    t_ids = jnp.stack([pos_t, neg_t], axis=1).reshape(-1)
    r_ids = jnp.stack([pos_r, neg_r], axis=1).reshape(-1)

    mesh = plsc.VectorSubcoreMesh(core_axis_name="c", subcore_axis_name="s")
    run = functools.partial(
        pl.kernel,
        out_type=jax.ShapeDtypeStruct((_NW, 16), jnp.float32),
        mesh=mesh,
        scratch_types=[
            pltpu.VMEM((_JPW,), jnp.int32),
            pltpu.VMEM((_JPW,), jnp.int32),
            pltpu.VMEM((_JPW,), jnp.int32),
            pltpu.VMEM((_CH, D * D), jnp.float32),
            pltpu.VMEM((_CH, D), jnp.float32),
            pltpu.VMEM((_CH, D), jnp.float32),
            pltpu.VMEM((_CH, D), jnp.float32),
            pltpu.VMEM((16,), jnp.float32),
            pltpu.SemaphoreType.DMA,
            pltpu.SemaphoreType.DMA,
            pltpu.SemaphoreType.DMA,
        ],
    )(_sc_body)
    return run(h_ids, t_ids, r_ids, ent_emb, rel_emb, transfer)


def kernel(x, ent_emb, rel_emb, transfer, l1_flag):
    part = _run_sc(x, ent_emb, rel_emb, transfer)
    loss1 = jnp.sum(part[:, 0])
    loss2 = jnp.sum(part[:, 1])
    return jnp.where(l1_flag, loss1, loss2)


# speedup vs baseline: 1.5057x; 1.0337x over previous
import functools

import jax
import jax.numpy as jnp
from jax import lax
from jax.experimental import pallas as pl
from jax.experimental.pallas import tpu as pltpu
from jax.experimental.pallas import tpu_sc as plsc

D = 64
B = 4096
MARGIN = 1.0

_NC = 2
_NS = 16
_NW = _NC * _NS
_JOBS = 2 * B
_JPW = _JOBS // _NW
_CH = 16
_NCHUNK = _JPW // _CH


def _allsum16(v):
    iota = lax.iota(jnp.int32, 16)
    for sh in (1, 2, 4, 8):
        v = v + v.at[iota ^ sh].get(mode="promise_in_bounds")
    return v


def _rsqrt16(x):
    i = lax.bitcast_convert_type(x, jnp.int32)
    y = lax.bitcast_convert_type(jnp.int32(0x5F3759DF) - (i >> 1), jnp.float32)
    for _ in range(3):
        y = y * (1.5 - 0.5 * x * y * y)
    return y


def _sc_body(hid_hbm, tid_hbm, rid_hbm, ent_hbm, rel_hbm, tr_hbm, out_hbm,
             hid_v, tid_v, rid_v, m_v, h_v, t_v, r_v, acc_v, sem_a, sem_s):
    wid = lax.axis_index("s") * _NC + lax.axis_index("c")
    base = wid * _JPW
    pltpu.sync_copy(hid_hbm.at[pl.ds(base, _JPW)], hid_v)
    pltpu.sync_copy(tid_hbm.at[pl.ds(base, _JPW)], tid_v)
    pltpu.sync_copy(rid_hbm.at[pl.ds(base, _JPW)], rid_v)

    splats = [jnp.full((16,), i, jnp.int32) for i in range(16)]

    def job_norms(acc):
        ss = acc[0] * acc[0] + acc[1] * acc[1] + acc[2] * acc[2] + acc[3] * acc[3]
        return _rsqrt16(jnp.maximum(_allsum16(ss), 1e-12))

    def job_scores(j, ah, at):
        inv_h = job_norms(ah)
        inv_t = job_norms(at)
        s_abs = jnp.zeros((16,), jnp.float32)
        s_sq = jnp.zeros((16,), jnp.float32)
        for k in range(4):
            rk = r_v[j, pl.ds(16 * k, 16)]
            dk = inv_h * ah[k] + rk - inv_t * at[k]
            s_abs = s_abs + jnp.abs(dk)
            s_sq = s_sq + dk * dk
        return _allsum16(s_abs), _allsum16(s_sq)

    def pair_body(p, pc):
        acc1, acc2 = pc
        j0 = 2 * p
        j1 = j0 + 1
        aph = [jnp.zeros((16,), jnp.float32) for _ in range(4)]
        apt = [jnp.zeros((16,), jnp.float32) for _ in range(4)]
        anh = [jnp.zeros((16,), jnp.float32) for _ in range(4)]
        ant = [jnp.zeros((16,), jnp.float32) for _ in range(4)]
        for d16 in range(4):
            hpv = h_v[j0, pl.ds(d16 * 16, 16)]
            hnv = h_v[j1, pl.ds(d16 * 16, 16)]
            tpv = t_v[j0, pl.ds(d16 * 16, 16)]
            tnv = t_v[j1, pl.ds(d16 * 16, 16)]
            for dd in range(16):
                hp = hpv.at[splats[dd]].get(mode="promise_in_bounds")
                hn = hnv.at[splats[dd]].get(mode="promise_in_bounds")
                tp = tpv.at[splats[dd]].get(mode="promise_in_bounds")
                tn = tnv.at[splats[dd]].get(mode="promise_in_bounds")
                off = (d16 * 16 + dd) * D
                for k in range(4):
                    mp = m_v[j0, pl.ds(off + 16 * k, 16)]
                    mn = m_v[j1, pl.ds(off + 16 * k, 16)]
                    aph[k] = aph[k] + hp * mp
                    apt[k] = apt[k] + tp * mp
                    anh[k] = anh[k] + hn * mn
                    ant[k] = ant[k] + tn * mn
        p1, p2 = job_scores(j0, aph, apt)
        n1, n2 = job_scores(j1, anh, ant)
        acc1 = acc1 + jnp.maximum(p1 - n1 + MARGIN, 0.0)
        acc2 = acc2 + jnp.maximum(p2 - n2 + MARGIN, 0.0)
        return acc1, acc2

    def chunk_body(c, carry):
        cb = c * _CH
        hidx = hid_v[pl.ds(cb, _CH)]
        tidx = tid_v[pl.ds(cb, _CH)]
        ridx = rid_v[pl.ds(cb, _CH)]
        cps = [pltpu.async_copy(tr_hbm.at[ridx], m_v, sem_a)]
        for jj in range(_CH):
            cps.append(pltpu.async_copy(ent_hbm.at[hidx[jj]], h_v.at[jj], sem_s))
            cps.append(pltpu.async_copy(ent_hbm.at[tidx[jj]], t_v.at[jj], sem_s))
            cps.append(pltpu.async_copy(rel_hbm.at[ridx[jj]], r_v.at[jj], sem_s))
        for cp in cps:
            cp.wait()
        return lax.fori_loop(0, _CH // 2, pair_body, carry)

    zero16 = jnp.zeros((16,), jnp.float32)
    acc1, acc2 = lax.fori_loop(0, _NCHUNK, chunk_body, (zero16, zero16))

    lane = lax.iota(jnp.int32, 16)
    res = jnp.where(lane == 0, acc1, jnp.where(lane == 1, acc2, 0.0))
    acc_v[...] = res
    pltpu.sync_copy(acc_v, out_hbm.at[wid])


def _run_sc(x, ent_emb, rel_emb, transfer):
    pos_h, pos_t, pos_r = x[:, 0], x[:, 1], x[:, 2]
    neg_h, neg_t, neg_r = x[:, 3], x[:, 4], x[:, 5]
    h_ids = jnp.stack([pos_h, neg_h], axis=1).reshape(-1)
    t_ids = jnp.stack([pos_t, neg_t], axis=1).reshape(-1)
    r_ids = jnp.stack([pos_r, neg_r], axis=1).reshape(-1)

    mesh = plsc.VectorSubcoreMesh(core_axis_name="c", subcore_axis_name="s")
    run = functools.partial(
        pl.kernel,
        out_type=jax.ShapeDtypeStruct((_NW, 16), jnp.float32),
        mesh=mesh,
        scratch_types=[
            pltpu.VMEM((_JPW,), jnp.int32),
            pltpu.VMEM((_JPW,), jnp.int32),
            pltpu.VMEM((_JPW,), jnp.int32),
            pltpu.VMEM((_CH, D * D), jnp.float32),
            pltpu.VMEM((_CH, D), jnp.float32),
            pltpu.VMEM((_CH, D), jnp.float32),
            pltpu.VMEM((_CH, D), jnp.float32),
            pltpu.VMEM((16,), jnp.float32),
            pltpu.SemaphoreType.DMA,
            pltpu.SemaphoreType.DMA,
        ],
    )(_sc_body)
    return run(h_ids, t_ids, r_ids, ent_emb, rel_emb, transfer)


def kernel(x, ent_emb, rel_emb, transfer, l1_flag):
    part = _run_sc(x, ent_emb, rel_emb, transfer)
    loss1 = jnp.sum(part[:, 0])
    loss2 = jnp.sum(part[:, 1])
    return jnp.where(l1_flag, loss1, loss2)
